# Initial kernel scaffold; baseline (speedup 1.0000x reference)
import jax
import jax.numpy as jnp
from jax.experimental import pallas as pl


def kernel(h, edge_index, edge_attr, W_e1, b_e1, W_e2, b_e2, W_n1, b_n1, W_n2, b_n2):
    raise NotImplementedError("write your pallas kernel here")


# speedup vs baseline: 2.0647x; 2.0647x over previous
import functools

import jax
import jax.numpy as jnp
from jax import lax
from jax.experimental import pallas as pl
from jax.experimental.pallas import tpu as pltpu
from jax.experimental.pallas import tpu_sc as plsc

_sds = jax.ShapeDtypeStruct

N = 10000
E = 160000
D = 256
DE = 16

NC = 2
NS = 16
NW = NC * NS

GCH = 80
EPW = 5040
EPAD = EPW * NW

SCH = 80
EPT = E // NS
SCHUNKS = EPT // SCH
NPAD = 10240
DH = D // 2
ROWS_PT = NPAD // NS

NBLK = 1000
EBLK = 2000


def _silu(x):
    return x * (1.0 / (1.0 + jnp.exp(-x)))


def _pre_body(h_ref, ws_ref, wt_ref, os_ref, ot_ref):
    hb = h_ref[...]
    os_ref[...] = jnp.dot(hb, ws_ref[...], preferred_element_type=jnp.float32)
    ot_ref[...] = jnp.dot(hb, wt_ref[...], preferred_element_type=jnp.float32)


def _edge_body(src_ref, tgt_ref, ea_ref, wee_ref, b1_ref, w2_ref, b2_ref,
               mij_ref):
    x = (src_ref[...] + tgt_ref[...]
         + jnp.dot(ea_ref[...], wee_ref[...],
                   preferred_element_type=jnp.float32)
         + b1_ref[...])
    t = _silu(x)
    y = jnp.dot(t, w2_ref[...], preferred_element_type=jnp.float32) + b2_ref[...]
    mij_ref[...] = _silu(y)


def _node_body(h_ref, agg_ref, w1h_ref, w1a_ref, b1_ref, w2_ref, b2_ref,
               o_ref):
    hb = h_ref[...]
    x = (jnp.dot(hb, w1h_ref[...], preferred_element_type=jnp.float32)
         + jnp.dot(agg_ref[...], w1a_ref[...],
                   preferred_element_type=jnp.float32)
         + b1_ref[...])
    t = _silu(x)
    o_ref[...] = (hb + jnp.dot(t, w2_ref[...],
                               preferred_element_type=jnp.float32)
                  + b2_ref[...])


def _gather_call(psrc, ptgt, rowp, colp):
    mesh = plsc.VectorSubcoreMesh(core_axis_name="c", subcore_axis_name="s")

    @functools.partial(
        pl.kernel,
        out_type=(_sds((EPAD, D), jnp.float32), _sds((EPAD, D), jnp.float32)),
        mesh=mesh,
        scratch_types=[
            pltpu.VMEM((GCH,), jnp.int32),
            pltpu.VMEM((GCH,), jnp.int32),
            pltpu.VMEM((GCH, D), jnp.float32),
            pltpu.VMEM((GCH, D), jnp.float32),
            pltpu.SemaphoreType.DMA,
            pltpu.SemaphoreType.DMA,
        ],
    )
    def gather_k(psrc_h, ptgt_h, rowp_h, colp_h, osrc_h, otgt_h,
                 idx_r, idx_c, buf_s, buf_t, sem_s, sem_t):
        c = lax.axis_index("c")
        s = lax.axis_index("s")
        base = (s * NC + c) * EPW

        def chunk(i, carry):
            off = base + i * GCH
            pltpu.sync_copy(rowp_h.at[pl.ds(off, GCH)], idx_r)
            pltpu.sync_copy(colp_h.at[pl.ds(off, GCH)], idx_c)
            cp1 = pltpu.async_copy(psrc_h.at[idx_r], buf_s, sem_s)
            cp2 = pltpu.async_copy(ptgt_h.at[idx_c], buf_t, sem_t)
            cp1.wait()
            cp2.wait()
            pltpu.sync_copy(buf_s, osrc_h.at[pl.ds(off, GCH)])
            pltpu.sync_copy(buf_t, otgt_h.at[pl.ds(off, GCH)])
            return carry

        lax.fori_loop(0, EPW // GCH, chunk, 0)

    return gather_k(psrc, ptgt, rowp, colp)


def _scatter_call(mij, row, zrows):
    mesh = plsc.VectorSubcoreMesh(core_axis_name="c", subcore_axis_name="s")

    @functools.partial(
        pl.kernel,
        out_type=_sds((NPAD, D), jnp.float32),
        mesh=mesh,
        scratch_types=[
            pltpu.VMEM((SCH,), jnp.int32),
            pltpu.VMEM((SCH, DH), jnp.float32),
            pltpu.VMEM_SHARED((NPAD, DH), jnp.float32),
        ],
    )
    def scatter_k(mij_h, row_h, zrows_h, agg_h, idx_v, mbuf, acc):
        c = lax.axis_index("c")
        s = lax.axis_index("s")
        pltpu.sync_copy(zrows_h, acc.at[pl.ds(s * ROWS_PT, ROWS_PT)])
        plsc.subcore_barrier()
        base = s * EPT

        def run_half(col0):
            def chunk(i, carry):
                off = base + i * SCH
                pltpu.sync_copy(row_h.at[pl.ds(off, SCH)], idx_v)
                pltpu.sync_copy(mij_h.at[pl.ds(off, SCH), pl.ds(col0, DH)],
                                mbuf)
                pltpu.sync_copy(mbuf, acc.at[idx_v], add=True)
                return carry

            lax.fori_loop(0, SCHUNKS, chunk, 0)
            plsc.subcore_barrier()
            pltpu.sync_copy(
                acc.at[pl.ds(s * ROWS_PT, ROWS_PT)],
                agg_h.at[pl.ds(s * ROWS_PT, ROWS_PT), pl.ds(col0, DH)])

        @pl.when(c == 0)
        def _():
            run_half(0)

        @pl.when(c == 1)
        def _():
            run_half(DH)

    return scatter_k(mij, row, zrows)


def kernel(h, edge_index, edge_attr, W_e1, b_e1, W_e2, b_e2,
           W_n1, b_n1, W_n2, b_n2):
    f32 = jnp.float32
    row = edge_index[0].astype(jnp.int32)
    col = edge_index[1].astype(jnp.int32)
    pad = jnp.zeros((EPAD - E,), jnp.int32)
    rowp = jnp.concatenate([row, pad])
    colp = jnp.concatenate([col, pad])

    p_src, p_tgt = pl.pallas_call(
        _pre_body,
        grid=(N // NBLK,),
        in_specs=[
            pl.BlockSpec((NBLK, D), lambda i: (i, 0)),
            pl.BlockSpec((D, D), lambda i: (0, 0)),
            pl.BlockSpec((D, D), lambda i: (0, 0)),
        ],
        out_specs=[pl.BlockSpec((NBLK, D), lambda i: (i, 0))] * 2,
        out_shape=[_sds((N, D), f32)] * 2,
    )(h, W_e1[:D], W_e1[D:2 * D])

    g_src, g_tgt = _gather_call(p_src, p_tgt, rowp, colp)

    mij = pl.pallas_call(
        _edge_body,
        grid=(E // EBLK,),
        in_specs=[
            pl.BlockSpec((EBLK, D), lambda i: (i, 0)),
            pl.BlockSpec((EBLK, D), lambda i: (i, 0)),
            pl.BlockSpec((EBLK, DE), lambda i: (i, 0)),
            pl.BlockSpec((DE, D), lambda i: (0, 0)),
            pl.BlockSpec((1, D), lambda i: (0, 0)),
            pl.BlockSpec((D, D), lambda i: (0, 0)),
            pl.BlockSpec((1, D), lambda i: (0, 0)),
        ],
        out_specs=pl.BlockSpec((EBLK, D), lambda i: (i, 0)),
        out_shape=_sds((E, D), f32),
    )(g_src, g_tgt, edge_attr, W_e1[2 * D:], b_e1.reshape(1, D),
      W_e2, b_e2.reshape(1, D))

    zrows = jnp.zeros((ROWS_PT, DH), f32)
    agg = _scatter_call(mij, row, zrows)

    h_out = pl.pallas_call(
        _node_body,
        grid=(N // NBLK,),
        in_specs=[
            pl.BlockSpec((NBLK, D), lambda i: (i, 0)),
            pl.BlockSpec((NBLK, D), lambda i: (i, 0)),
            pl.BlockSpec((D, D), lambda i: (0, 0)),
            pl.BlockSpec((D, D), lambda i: (0, 0)),
            pl.BlockSpec((1, D), lambda i: (0, 0)),
            pl.BlockSpec((D, D), lambda i: (0, 0)),
            pl.BlockSpec((1, D), lambda i: (0, 0)),
        ],
        out_specs=pl.BlockSpec((NBLK, D), lambda i: (i, 0)),
        out_shape=_sds((N, D), f32),
    )(h, agg, W_n1[:D], W_n1[D:], b_n1.reshape(1, D),
      W_n2, b_n2.reshape(1, D))

    return (h_out, mij)
